# Initial kernel scaffold; baseline (speedup 1.0000x reference)
import jax
import jax.numpy as jnp
from jax.experimental import pallas as pl


def kernel(users, pos, neg, E, W1, b1, W2, b2, W3, b3):
    raise NotImplementedError("write your pallas kernel here")


# speedup vs baseline: 38.5273x; 38.5273x over previous
import functools

import numpy as np

import jax
import jax.numpy as jnp
from jax import lax
from jax.experimental import pallas as pl
from jax.experimental.pallas import tpu as pltpu
from jax.experimental.pallas import tpu_sc as plsc

N = 50000
D = 64
H = 32
NPAD = 50048
E2 = 1600000
K = 100
E2K = E2 // K
TILES = 16
ROWS_PER_TILE_E = E2K // TILES
NJ = 8
OUTER_E = ROWS_PER_TILE_E // NJ
WB = NPAD // TILES

NE = 800000
NEP = 819200
DF = 4 * D
K2 = 64
NEK = NEP // K2
NW = 32
CH_PER_W = NEK // NW
OB = 16
OUTER_S = CH_PER_W // OB
UNROLL = 8

_I0 = np.int32(0)
ROWBLK = 400
GRID_TC = N // ROWBLK

def _gcn_sc_body(y2, dg, ss, zz, out, gidx, sidx, rows0, rows1, acc, sem0, sem1):
    c = lax.axis_index("c")
    s = lax.axis_index("s")

    @pl.when(s == 0)
    def _():
        pltpu.sync_copy(zz, acc)

    plsc.subcore_barrier()

    rows = (rows0, rows1)
    sems = (sem0, sem1)

    def outer(i, carry):
        row0 = s * ROWS_PER_TILE_E + i * NJ
        pltpu.sync_copy(dg.at[pl.ds(c * E2K + row0, NJ)], gidx)
        pltpu.sync_copy(ss.at[pl.ds(row0, NJ)], sidx)
        d = [None, None]
        d[0] = pltpu.async_copy(y2.at[gidx.at[jnp.int32(0)]], rows[0], sems[0])
        for j in range(NJ):
            b = j & 1
            d[b].wait()
            if j + 1 < NJ:
                d[1 - b] = pltpu.async_copy(
                    y2.at[gidx.at[jnp.int32(j + 1)]], rows[1 - b], sems[1 - b])
            pltpu.sync_copy(rows[b], acc.at[sidx.at[jnp.int32(j)]], add=True)
        return carry

    lax.fori_loop(jnp.int32(0), jnp.int32(OUTER_E), outer, jnp.int32(0))
    plsc.subcore_barrier()
    pltpu.sync_copy(acc.at[pl.ds(s * WB, WB)],
                    out.at[pl.ds(c * NPAD + s * WB, WB)])


@functools.cache
def _gcn_call():
    mesh = plsc.VectorSubcoreMesh(core_axis_name="c", subcore_axis_name="s")
    return pl.kernel(
        _gcn_sc_body,
        out_type=jax.ShapeDtypeStruct((2 * NPAD, H), jnp.float32),
        mesh=mesh,
        compiler_params=pltpu.CompilerParams(use_tc_tiling_on_sc=False, needs_layout_passes=False),
        scratch_types=[
            pltpu.VMEM((NJ, K), jnp.int32),
            pltpu.VMEM((NJ, K), jnp.int32),
            pltpu.VMEM((K, H), jnp.float32),
            pltpu.VMEM((K, H), jnp.float32),
            pltpu.VMEM_SHARED((NPAD, H), jnp.float32),
            pltpu.SemaphoreType.DMA,
            pltpu.SemaphoreType.DMA,
        ],
    )


def _sim_sc_body(ef, ui, pi, ni, ps, ns,
                 uix, pix, nix, ub, pb, nb2, pob, nob, semu, semp, semn):
    c = lax.axis_index("c")
    s = lax.axis_index("s")
    w = s * 2 + c
    iota = lax.iota(jnp.int32, 16)

    def outer(o, carry):
        r0 = w * CH_PER_W + o * OB
        pltpu.sync_copy(ui.at[pl.ds(r0, OB)], uix)
        pltpu.sync_copy(pi.at[pl.ds(r0, OB)], pix)
        pltpu.sync_copy(ni.at[pl.ds(r0, OB)], nix)

        def inner(k, carry2):
            du = pltpu.async_copy(ef.at[uix.at[k]], ub, semu)
            dp = pltpu.async_copy(ef.at[pix.at[k]], pb, semp)
            dn = pltpu.async_copy(ef.at[nix.at[k]], nb2, semn)
            du.wait()
            dp.wait()
            dn.wait()
            for g in range(K2 // 16):
                rowv = iota + jnp.int32(g * 16)

                def jloop(t, car):
                    jv, accp, accn = car
                    for u in range(UNROLL):
                        col = jv + jnp.int32(u)
                        uv = plsc.load_gather(ub, [rowv, col])
                        pv = plsc.load_gather(pb, [rowv, col])
                        nv = plsc.load_gather(nb2, [rowv, col])
                        accp = accp + uv * pv
                        accn = accn + uv * nv
                    return (jv + jnp.int32(UNROLL), accp, accn)

                z = jnp.zeros((16,), jnp.float32)
                j0 = jnp.zeros((16,), jnp.int32)
                _, accp, accn = lax.fori_loop(
                    jnp.int32(0), jnp.int32(DF // UNROLL), jloop, (j0, z, z))
                pob[pl.ds(g * 16, 16)] = accp
                nob[pl.ds(g * 16, 16)] = accn
            base = (r0 + k) * K2
            pltpu.sync_copy(pob, ps.at[pl.ds(base, K2)])
            pltpu.sync_copy(nob, ns.at[pl.ds(base, K2)])
            return carry2

        lax.fori_loop(jnp.int32(0), jnp.int32(OB), inner, jnp.int32(0))
        return carry

    lax.fori_loop(jnp.int32(0), jnp.int32(OUTER_S), outer, jnp.int32(0))


@functools.cache
def _sim_call():
    mesh = plsc.VectorSubcoreMesh(core_axis_name="c", subcore_axis_name="s")
    return pl.kernel(
        _sim_sc_body,
        out_type=(jax.ShapeDtypeStruct((NEP,), jnp.float32),
                  jax.ShapeDtypeStruct((NEP,), jnp.float32)),
        mesh=mesh,
        compiler_params=pltpu.CompilerParams(use_tc_tiling_on_sc=False, needs_layout_passes=False),
        scratch_types=[
            pltpu.VMEM((OB, K2), jnp.int32),
            pltpu.VMEM((OB, K2), jnp.int32),
            pltpu.VMEM((OB, K2), jnp.int32),
            pltpu.VMEM((K2, DF), jnp.float32),
            pltpu.VMEM((K2, DF), jnp.float32),
            pltpu.VMEM((K2, DF), jnp.float32),
            pltpu.VMEM((K2,), jnp.float32),
            pltpu.VMEM((K2,), jnp.float32),
            pltpu.SemaphoreType.DMA,
            pltpu.SemaphoreType.DMA,
            pltpu.SemaphoreType.DMA,
        ],
    )


def _t0_body(x_ref, w_ref, dinv_ref, y2_ref):
    xw = jnp.dot(x_ref[...], w_ref[...], preferred_element_type=jnp.float32)
    y = xw * dinv_ref[...]
    y2_ref[0] = y[:, :H]
    y2_ref[1] = y[:, H:]


_t0 = pl.pallas_call(
    _t0_body,
    grid=(GRID_TC,),
    in_specs=[
        pl.BlockSpec((ROWBLK, D), lambda i: (i, _I0)),
        pl.BlockSpec((D, D), lambda i: (_I0, _I0)),
        pl.BlockSpec((ROWBLK, 1), lambda i: (i, _I0)),
    ],
    out_specs=pl.BlockSpec((2, ROWBLK, H), lambda i: (_I0, i, _I0)),
    out_shape=jax.ShapeDtypeStruct((2, N, H), jnp.float32),
)


def _post_layer(acc_ref, y_ref, dinv_ref, b_ref):
    t0 = dinv_ref[...] * (acc_ref[0] + y_ref[0]) + b_ref[0, :H]
    t1 = dinv_ref[...] * (acc_ref[1] + y_ref[1]) + b_ref[0, H:]
    u0 = jnp.where(t0 > 0, t0, 0.01 * t0)
    u1 = jnp.where(t1 > 0, t1, 0.01 * t1)
    nsq = (jnp.sum(u0 * u0, axis=1, keepdims=True)
           + jnp.sum(u1 * u1, axis=1, keepdims=True))
    scale = 1.0 / jnp.maximum(jnp.sqrt(nsq), 1e-12)
    return u0 * scale, u1 * scale


def _fmid_body(acc_ref, y_ref, dinv_ref, b_ref, w_ref, e_ref, yn_ref):
    e0, e1 = _post_layer(acc_ref, y_ref, dinv_ref, b_ref)
    e_ref[:, :H] = e0
    e_ref[:, H:] = e1
    e = jnp.concatenate([e0, e1], axis=1)
    yn = jnp.dot(e, w_ref[...], preferred_element_type=jnp.float32) * dinv_ref[...]
    yn_ref[0] = yn[:, :H]
    yn_ref[1] = yn[:, H:]


_fmid = pl.pallas_call(
    _fmid_body,
    grid=(GRID_TC,),
    in_specs=[
        pl.BlockSpec((2, ROWBLK, H), lambda i: (_I0, i, _I0)),
        pl.BlockSpec((2, ROWBLK, H), lambda i: (_I0, i, _I0)),
        pl.BlockSpec((ROWBLK, 1), lambda i: (i, _I0)),
        pl.BlockSpec((1, D), lambda i: (_I0, _I0)),
        pl.BlockSpec((D, D), lambda i: (_I0, _I0)),
    ],
    out_specs=[
        pl.BlockSpec((ROWBLK, D), lambda i: (i, _I0)),
        pl.BlockSpec((2, ROWBLK, H), lambda i: (_I0, i, _I0)),
    ],
    out_shape=[
        jax.ShapeDtypeStruct((N, D), jnp.float32),
        jax.ShapeDtypeStruct((2, N, H), jnp.float32),
    ],
)


def _flast_body(acc_ref, y_ref, dinv_ref, b_ref, e_ref):
    e0, e1 = _post_layer(acc_ref, y_ref, dinv_ref, b_ref)
    e_ref[:, :H] = e0
    e_ref[:, H:] = e1


_flast = pl.pallas_call(
    _flast_body,
    grid=(GRID_TC,),
    in_specs=[
        pl.BlockSpec((2, ROWBLK, H), lambda i: (_I0, i, _I0)),
        pl.BlockSpec((2, ROWBLK, H), lambda i: (_I0, i, _I0)),
        pl.BlockSpec((ROWBLK, 1), lambda i: (i, _I0)),
        pl.BlockSpec((1, D), lambda i: (_I0, _I0)),
    ],
    out_specs=pl.BlockSpec((ROWBLK, D), lambda i: (i, _I0)),
    out_shape=jax.ShapeDtypeStruct((N, D), jnp.float32),
)


def kernel(users, pos, neg, E, W1, b1, W2, b2, W3, b3):
    E = E.astype(jnp.float32)
    W1, b1, W2, b2, W3, b3 = (t.astype(jnp.float32)
                              for t in (W1, b1, W2, b2, W3, b3))
    u32 = users.astype(jnp.uint32)
    p32 = pos.astype(jnp.uint32)
    nn = jnp.uint32(N)
    eid = jnp.sort(jnp.concatenate([u32 * nn + p32, p32 * nn + u32]))
    keep = jnp.concatenate([jnp.ones((1,), bool), eid[1:] != eid[:-1]])
    src = (eid // nn).astype(jnp.int32)
    dst = (eid % nn).astype(jnp.int32)
    srcS = jnp.where(keep, src, N).astype(jnp.int32)
    dstG = jnp.where(keep, dst, 0).astype(jnp.int32)
    deg = jax.ops.segment_sum(keep.astype(jnp.float32), src, num_segments=N) + 1.0
    dinv = lax.rsqrt(deg)
    dinv2 = dinv[:, None]

    dstG2 = jnp.concatenate([dstG, dstG + N]).reshape(2 * E2K, K)
    srcS2 = srcS.reshape(E2K, K)
    zz = jnp.zeros((NPAD, H), jnp.float32)
    b1r, b2r, b3r = (b.reshape(1, D) for b in (b1, b2, b3))

    def gcn(y2):
        acc = _gcn_call()(y2.reshape(2 * N, H), dstG2, srcS2, zz)
        return acc.reshape(2, NPAD, H)[:, :N, :]

    y2a = _t0(E, W1, dinv2)
    e1, y2b = _fmid(gcn(y2a), y2a, dinv2, b1r, W2)
    e2, y2c = _fmid(gcn(y2b), y2b, dinv2, b2r, W3)
    e3 = _flast(gcn(y2c), y2c, dinv2, b3r)

    ef = jnp.concatenate([E, e1, e2, e3], axis=1)
    padw = NEP - NE
    ui = jnp.pad(users.astype(jnp.int32), (0, padw)).reshape(NEK, K2)
    pi = jnp.pad(pos.astype(jnp.int32), (0, padw)).reshape(NEK, K2)
    ni = jnp.pad(neg.astype(jnp.int32), (0, padw)).reshape(NEK, K2)
    ps, ns = _sim_call()(ef, ui, pi, ni)
    return ps[:NE].astype(jnp.float64), ns[:NE].astype(jnp.float64)


# speedup vs baseline: 43.5441x; 1.1302x over previous
import functools

import numpy as np

import jax
import jax.numpy as jnp
from jax import lax
from jax.experimental import pallas as pl
from jax.experimental.pallas import tpu as pltpu
from jax.experimental.pallas import tpu_sc as plsc

N = 50000
D = 64
H = 32
NPAD = 50048
E2 = 1600000
K = 100
E2K = E2 // K
TILES = 16
ROWS_PER_TILE_E = E2K // TILES
NJ = 8
OUTER_E = ROWS_PER_TILE_E // NJ
WB = NPAD // TILES

NE = 800000
NEP = 819200
DF = 4 * D
K2 = 64
NEK = NEP // K2
NW = 32
CH_PER_W = NEK // NW
NBLK = 16
NPAIR = NBLK // 2
OUTER_S = CH_PER_W // NBLK

_I0 = np.int32(0)
ROWBLK = 400
GRID_TC = N // ROWBLK

def _gcn_sc_body(y2, dg, ss, zz, out, gidx, sidx, rows0, rows1, acc, sem0, sem1):
    c = lax.axis_index("c")
    s = lax.axis_index("s")

    @pl.when(s == 0)
    def _():
        pltpu.sync_copy(zz, acc)

    plsc.subcore_barrier()

    rows = (rows0, rows1)
    sems = (sem0, sem1)

    def outer(i, carry):
        row0 = s * ROWS_PER_TILE_E + i * NJ
        pltpu.sync_copy(dg.at[pl.ds(c * E2K + row0, NJ)], gidx)
        pltpu.sync_copy(ss.at[pl.ds(row0, NJ)], sidx)
        d = [None, None]
        d[0] = pltpu.async_copy(y2.at[gidx.at[jnp.int32(0)]], rows[0], sems[0])
        for j in range(NJ):
            b = j & 1
            d[b].wait()
            if j + 1 < NJ:
                d[1 - b] = pltpu.async_copy(
                    y2.at[gidx.at[jnp.int32(j + 1)]], rows[1 - b], sems[1 - b])
            pltpu.sync_copy(rows[b], acc.at[sidx.at[jnp.int32(j)]], add=True)
        return carry

    lax.fori_loop(jnp.int32(0), jnp.int32(OUTER_E), outer, jnp.int32(0))
    plsc.subcore_barrier()
    pltpu.sync_copy(acc.at[pl.ds(s * WB, WB)],
                    out.at[pl.ds(c * NPAD + s * WB, WB)])


@functools.cache
def _gcn_call():
    mesh = plsc.VectorSubcoreMesh(core_axis_name="c", subcore_axis_name="s")
    return pl.kernel(
        _gcn_sc_body,
        out_type=jax.ShapeDtypeStruct((2 * NPAD, H), jnp.float32),
        mesh=mesh,
        compiler_params=pltpu.CompilerParams(use_tc_tiling_on_sc=False, needs_layout_passes=False),
        scratch_types=[
            pltpu.VMEM((NJ, K), jnp.int32),
            pltpu.VMEM((NJ, K), jnp.int32),
            pltpu.VMEM((K, H), jnp.float32),
            pltpu.VMEM((K, H), jnp.float32),
            pltpu.VMEM_SHARED((NPAD, H), jnp.float32),
            pltpu.SemaphoreType.DMA,
            pltpu.SemaphoreType.DMA,
        ],
    )


def _sim_sc_body(ef, ui, pi, ni, ps, ns,
                 uix, pix, nix, ub0, pb0, nb0, ub1, pb1, nb1,
                 pob, nob, su0, sp0, sn0, su1, sp1, sn1):
    c = lax.axis_index("c")
    s = lax.axis_index("s")
    w = s * 2 + c
    iota = lax.iota(jnp.int32, 16)
    sets = ((ub0, pb0, nb0, su0, sp0, sn0), (ub1, pb1, nb1, su1, sp1, sn1))

    def gather(st, idxrow):
        bu, bp, bn, su, sp, sn = st
        pltpu.async_copy(ef.at[uix.at[idxrow]], bu, su)
        pltpu.async_copy(ef.at[pix.at[idxrow]], bp, sp)
        pltpu.async_copy(ef.at[nix.at[idxrow]], bn, sn)

    def waitset(st):
        bu, bp, bn, su, sp, sn = st
        z = jnp.int32(0)
        pltpu.make_async_copy(ef.at[uix.at[z]], bu, su).wait()
        pltpu.make_async_copy(ef.at[pix.at[z]], bp, sp).wait()
        pltpu.make_async_copy(ef.at[nix.at[z]], bn, sn).wait()

    def compute(st, slot):
        bu, bp, bn = st[0], st[1], st[2]
        for g in range(K2 // 16):
            rowv = iota + jnp.int32(g * 16)

            def jloop(t2, car):
                jv = car[0]
                a = list(car[1:])
                for u in range(8):
                    col = jv + jnp.int32(u)
                    uv = plsc.load_gather(bu, [rowv, col])
                    pv = plsc.load_gather(bp, [rowv, col])
                    nv = plsc.load_gather(bn, [rowv, col])
                    k = u & 3
                    a[k] = a[k] + uv * pv
                    a[4 + k] = a[4 + k] + uv * nv
                return (jv + jnp.int32(8),) + tuple(a)

            z = jnp.zeros((16,), jnp.float32)
            init = (jnp.zeros((16,), jnp.int32),) + (z,) * 8
            r = lax.fori_loop(jnp.int32(0), jnp.int32(DF // 8), jloop, init)
            accp = (r[1] + r[2]) + (r[3] + r[4])
            accn = (r[5] + r[6]) + (r[7] + r[8])
            off = slot * jnp.int32(K2) + jnp.int32(g * 16)
            pob[pl.ds(off, 16)] = accp
            nob[pl.ds(off, 16)] = accn

    def outer(b, carry):
        r0 = w * CH_PER_W + b * NBLK
        pltpu.sync_copy(ui.at[pl.ds(r0, NBLK)], uix)
        pltpu.sync_copy(pi.at[pl.ds(r0, NBLK)], pix)
        pltpu.sync_copy(ni.at[pl.ds(r0, NBLK)], nix)
        gather(sets[0], jnp.int32(0))

        def pair(t, carry2):
            t2 = t * 2
            gather(sets[1], t2 + 1)
            waitset(sets[0])
            compute(sets[0], t2)

            @pl.when(t < NPAIR - 1)
            def _():
                gather(sets[0], t2 + 2)

            waitset(sets[1])
            compute(sets[1], t2 + 1)
            return carry2

        lax.fori_loop(jnp.int32(0), jnp.int32(NPAIR), pair, jnp.int32(0))
        base = r0 * K2
        pltpu.sync_copy(pob, ps.at[pl.ds(base, NBLK * K2)])
        pltpu.sync_copy(nob, ns.at[pl.ds(base, NBLK * K2)])
        return carry

    lax.fori_loop(jnp.int32(0), jnp.int32(OUTER_S), outer, jnp.int32(0))


@functools.cache
def _sim_call():
    mesh = plsc.VectorSubcoreMesh(core_axis_name="c", subcore_axis_name="s")
    return pl.kernel(
        _sim_sc_body,
        out_type=(jax.ShapeDtypeStruct((NEP,), jnp.float32),
                  jax.ShapeDtypeStruct((NEP,), jnp.float32)),
        mesh=mesh,
        compiler_params=pltpu.CompilerParams(use_tc_tiling_on_sc=False, needs_layout_passes=False),
        scratch_types=[
            pltpu.VMEM((NBLK, K2), jnp.int32),
            pltpu.VMEM((NBLK, K2), jnp.int32),
            pltpu.VMEM((NBLK, K2), jnp.int32),
            pltpu.VMEM((K2, DF), jnp.float32),
            pltpu.VMEM((K2, DF), jnp.float32),
            pltpu.VMEM((K2, DF), jnp.float32),
            pltpu.VMEM((K2, DF), jnp.float32),
            pltpu.VMEM((K2, DF), jnp.float32),
            pltpu.VMEM((K2, DF), jnp.float32),
            pltpu.VMEM((NBLK * K2,), jnp.float32),
            pltpu.VMEM((NBLK * K2,), jnp.float32),
            pltpu.SemaphoreType.DMA,
            pltpu.SemaphoreType.DMA,
            pltpu.SemaphoreType.DMA,
            pltpu.SemaphoreType.DMA,
            pltpu.SemaphoreType.DMA,
            pltpu.SemaphoreType.DMA,
        ],
    )


def _t0_body(x_ref, w_ref, dinv_ref, y2_ref):
    xw = jnp.dot(x_ref[...], w_ref[...], preferred_element_type=jnp.float32)
    y = xw * dinv_ref[...]
    y2_ref[0] = y[:, :H]
    y2_ref[1] = y[:, H:]


_t0 = pl.pallas_call(
    _t0_body,
    grid=(GRID_TC,),
    in_specs=[
        pl.BlockSpec((ROWBLK, D), lambda i: (i, _I0)),
        pl.BlockSpec((D, D), lambda i: (_I0, _I0)),
        pl.BlockSpec((ROWBLK, 1), lambda i: (i, _I0)),
    ],
    out_specs=pl.BlockSpec((2, ROWBLK, H), lambda i: (_I0, i, _I0)),
    out_shape=jax.ShapeDtypeStruct((2, N, H), jnp.float32),
)


def _post_layer(acc_ref, y_ref, dinv_ref, b_ref):
    t0 = dinv_ref[...] * (acc_ref[0] + y_ref[0]) + b_ref[0, :H]
    t1 = dinv_ref[...] * (acc_ref[1] + y_ref[1]) + b_ref[0, H:]
    u0 = jnp.where(t0 > 0, t0, 0.01 * t0)
    u1 = jnp.where(t1 > 0, t1, 0.01 * t1)
    nsq = (jnp.sum(u0 * u0, axis=1, keepdims=True)
           + jnp.sum(u1 * u1, axis=1, keepdims=True))
    scale = 1.0 / jnp.maximum(jnp.sqrt(nsq), 1e-12)
    return u0 * scale, u1 * scale


def _fmid_body(acc_ref, y_ref, dinv_ref, b_ref, w_ref, e_ref, yn_ref):
    e0, e1 = _post_layer(acc_ref, y_ref, dinv_ref, b_ref)
    e_ref[:, :H] = e0
    e_ref[:, H:] = e1
    e = jnp.concatenate([e0, e1], axis=1)
    yn = jnp.dot(e, w_ref[...], preferred_element_type=jnp.float32) * dinv_ref[...]
    yn_ref[0] = yn[:, :H]
    yn_ref[1] = yn[:, H:]


_fmid = pl.pallas_call(
    _fmid_body,
    grid=(GRID_TC,),
    in_specs=[
        pl.BlockSpec((2, ROWBLK, H), lambda i: (_I0, i, _I0)),
        pl.BlockSpec((2, ROWBLK, H), lambda i: (_I0, i, _I0)),
        pl.BlockSpec((ROWBLK, 1), lambda i: (i, _I0)),
        pl.BlockSpec((1, D), lambda i: (_I0, _I0)),
        pl.BlockSpec((D, D), lambda i: (_I0, _I0)),
    ],
    out_specs=[
        pl.BlockSpec((ROWBLK, D), lambda i: (i, _I0)),
        pl.BlockSpec((2, ROWBLK, H), lambda i: (_I0, i, _I0)),
    ],
    out_shape=[
        jax.ShapeDtypeStruct((N, D), jnp.float32),
        jax.ShapeDtypeStruct((2, N, H), jnp.float32),
    ],
)


def _flast_body(acc_ref, y_ref, dinv_ref, b_ref, e_ref):
    e0, e1 = _post_layer(acc_ref, y_ref, dinv_ref, b_ref)
    e_ref[:, :H] = e0
    e_ref[:, H:] = e1


_flast = pl.pallas_call(
    _flast_body,
    grid=(GRID_TC,),
    in_specs=[
        pl.BlockSpec((2, ROWBLK, H), lambda i: (_I0, i, _I0)),
        pl.BlockSpec((2, ROWBLK, H), lambda i: (_I0, i, _I0)),
        pl.BlockSpec((ROWBLK, 1), lambda i: (i, _I0)),
        pl.BlockSpec((1, D), lambda i: (_I0, _I0)),
    ],
    out_specs=pl.BlockSpec((ROWBLK, D), lambda i: (i, _I0)),
    out_shape=jax.ShapeDtypeStruct((N, D), jnp.float32),
)


def kernel(users, pos, neg, E, W1, b1, W2, b2, W3, b3):
    E = E.astype(jnp.float32)
    W1, b1, W2, b2, W3, b3 = (t.astype(jnp.float32)
                              for t in (W1, b1, W2, b2, W3, b3))
    u32 = users.astype(jnp.uint32)
    p32 = pos.astype(jnp.uint32)
    nn = jnp.uint32(N)
    eid = jnp.sort(jnp.concatenate([u32 * nn + p32, p32 * nn + u32]))
    keep = jnp.concatenate([jnp.ones((1,), bool), eid[1:] != eid[:-1]])
    src = (eid // nn).astype(jnp.int32)
    dst = (eid % nn).astype(jnp.int32)
    srcS = jnp.where(keep, src, N).astype(jnp.int32)
    dstG = jnp.where(keep, dst, 0).astype(jnp.int32)
    deg = jax.ops.segment_sum(keep.astype(jnp.float32), src, num_segments=N) + 1.0
    dinv = lax.rsqrt(deg)
    dinv2 = dinv[:, None]

    dstG2 = jnp.concatenate([dstG, dstG + N]).reshape(2 * E2K, K)
    srcS2 = srcS.reshape(E2K, K)
    zz = jnp.zeros((NPAD, H), jnp.float32)
    b1r, b2r, b3r = (b.reshape(1, D) for b in (b1, b2, b3))

    def gcn(y2):
        acc = _gcn_call()(y2.reshape(2 * N, H), dstG2, srcS2, zz)
        return acc.reshape(2, NPAD, H)[:, :N, :]

    y2a = _t0(E, W1, dinv2)
    e1, y2b = _fmid(gcn(y2a), y2a, dinv2, b1r, W2)
    e2, y2c = _fmid(gcn(y2b), y2b, dinv2, b2r, W3)
    e3 = _flast(gcn(y2c), y2c, dinv2, b3r)

    ef = jnp.concatenate([E, e1, e2, e3], axis=1)
    padw = NEP - NE
    ui = jnp.pad(users.astype(jnp.int32), (0, padw)).reshape(NEK, K2)
    pi = jnp.pad(pos.astype(jnp.int32), (0, padw)).reshape(NEK, K2)
    ni = jnp.pad(neg.astype(jnp.int32), (0, padw)).reshape(NEK, K2)
    ps, ns = _sim_call()(ef, ui, pi, ni)
    return ps[:NE].astype(jnp.float64), ns[:NE].astype(jnp.float64)


# speedup vs baseline: 60.4128x; 1.3874x over previous
import functools

import numpy as np

import jax
import jax.numpy as jnp
from jax import lax
from jax.experimental import pallas as pl
from jax.experimental.pallas import tpu as pltpu
from jax.experimental.pallas import tpu_sc as plsc

N = 50000
D = 64
H = 32
NPAD = 50048
E2 = 1600000
K = 100
E2K = E2 // K
TILES = 16
ROWS_PER_TILE_E = E2K // TILES
NJ = 8
OUTER_E = ROWS_PER_TILE_E // NJ
WB = NPAD // TILES

NE = 800000
NEP = 819200
DF = 4 * D
K2 = 64
NEK = NEP // K2
NW = 32
CH_PER_W = NEK // NW
DFP = DF // 2
NBLK = 16
NPAIR = NBLK // 2
OUTER_S = CH_PER_W // NBLK

_I0 = np.int32(0)
ROWBLK = 400
GRID_TC = N // ROWBLK

def _gcn_sc_body(y2, dg, ss, zz, out, gidx, sidx, rows0, rows1, acc, sem0, sem1):
    c = lax.axis_index("c")
    s = lax.axis_index("s")

    @pl.when(s == 0)
    def _():
        pltpu.sync_copy(zz, acc)

    plsc.subcore_barrier()

    rows = (rows0, rows1)
    sems = (sem0, sem1)

    def outer(i, carry):
        row0 = s * ROWS_PER_TILE_E + i * NJ
        pltpu.sync_copy(dg.at[pl.ds(c * E2K + row0, NJ)], gidx)
        pltpu.sync_copy(ss.at[pl.ds(row0, NJ)], sidx)
        d = [None, None]
        d[0] = pltpu.async_copy(y2.at[gidx.at[jnp.int32(0)]], rows[0], sems[0])
        for j in range(NJ):
            b = j & 1
            d[b].wait()
            if j + 1 < NJ:
                d[1 - b] = pltpu.async_copy(
                    y2.at[gidx.at[jnp.int32(j + 1)]], rows[1 - b], sems[1 - b])
            pltpu.sync_copy(rows[b], acc.at[sidx.at[jnp.int32(j)]], add=True)
        return carry

    lax.fori_loop(jnp.int32(0), jnp.int32(OUTER_E), outer, jnp.int32(0))
    plsc.subcore_barrier()
    pltpu.sync_copy(acc.at[pl.ds(s * WB, WB)],
                    out.at[pl.ds(c * NPAD + s * WB, WB)])


@functools.cache
def _gcn_call():
    mesh = plsc.VectorSubcoreMesh(core_axis_name="c", subcore_axis_name="s")
    return pl.kernel(
        _gcn_sc_body,
        out_type=jax.ShapeDtypeStruct((2 * NPAD, H), jnp.float32),
        mesh=mesh,
        compiler_params=pltpu.CompilerParams(use_tc_tiling_on_sc=False, needs_layout_passes=False),
        scratch_types=[
            pltpu.VMEM((NJ, K), jnp.int32),
            pltpu.VMEM((NJ, K), jnp.int32),
            pltpu.VMEM((K, H), jnp.float32),
            pltpu.VMEM((K, H), jnp.float32),
            pltpu.VMEM_SHARED((NPAD, H), jnp.float32),
            pltpu.SemaphoreType.DMA,
            pltpu.SemaphoreType.DMA,
        ],
    )


def _sim_sc_body(ef, ui, pi, ni, ps, ns,
                 uix, pix, nix, ub0, pb0, nb0, ub1, pb1, nb1,
                 pob, nob, su0, sp0, sn0, su1, sp1, sn1):
    c = lax.axis_index("c")
    s = lax.axis_index("s")
    w = s * 2 + c
    iota = lax.iota(jnp.int32, 16)
    sets = ((ub0, pb0, nb0, su0, sp0, sn0), (ub1, pb1, nb1, su1, sp1, sn1))

    def gather(st, idxrow):
        bu, bp, bn, su, sp, sn = st
        pltpu.async_copy(ef.at[uix.at[idxrow]], bu, su)
        pltpu.async_copy(ef.at[pix.at[idxrow]], bp, sp)
        pltpu.async_copy(ef.at[nix.at[idxrow]], bn, sn)

    def waitset(st):
        bu, bp, bn, su, sp, sn = st
        z = jnp.int32(0)
        pltpu.make_async_copy(ef.at[uix.at[z]], bu, su).wait()
        pltpu.make_async_copy(ef.at[pix.at[z]], bp, sp).wait()
        pltpu.make_async_copy(ef.at[nix.at[z]], bn, sn).wait()

    def compute(st, slot):
        bu, bp, bn = st[0], st[1], st[2]
        for g in range(K2 // 16):
            rowv = iota + jnp.int32(g * 16)

            def jloop(t2, car):
                jv = car[0]
                a = list(car[1:])
                for u in range(8):
                    col = jv + jnp.int32(u)
                    uw = plsc.load_gather(bu, [rowv, col])
                    pw = plsc.load_gather(bp, [rowv, col])
                    nw = plsc.load_gather(bn, [rowv, col])
                    u0, u1 = plsc.unpack(plsc.bitcast(uw, jnp.bfloat16),
                                         format=plsc.PackFormat.INTERLEAVED)
                    p0, p1 = plsc.unpack(plsc.bitcast(pw, jnp.bfloat16),
                                         format=plsc.PackFormat.INTERLEAVED)
                    n0, n1 = plsc.unpack(plsc.bitcast(nw, jnp.bfloat16),
                                         format=plsc.PackFormat.INTERLEAVED)
                    k = u & 3
                    a[k] = a[k] + (u0 * p0 + u1 * p1)
                    a[4 + k] = a[4 + k] + (u0 * n0 + u1 * n1)
                return (jv + jnp.int32(8),) + tuple(a)

            z = jnp.zeros((16,), jnp.float32)
            init = (jnp.zeros((16,), jnp.int32),) + (z,) * 8
            r = lax.fori_loop(jnp.int32(0), jnp.int32(DFP // 8), jloop, init)
            accp = (r[1] + r[2]) + (r[3] + r[4])
            accn = (r[5] + r[6]) + (r[7] + r[8])
            off = slot * jnp.int32(K2) + jnp.int32(g * 16)
            pob[pl.ds(off, 16)] = accp
            nob[pl.ds(off, 16)] = accn

    def outer(b, carry):
        r0 = w * CH_PER_W + b * NBLK
        pltpu.sync_copy(ui.at[pl.ds(r0, NBLK)], uix)
        pltpu.sync_copy(pi.at[pl.ds(r0, NBLK)], pix)
        pltpu.sync_copy(ni.at[pl.ds(r0, NBLK)], nix)
        gather(sets[0], jnp.int32(0))

        def pair(t, carry2):
            t2 = t * 2
            gather(sets[1], t2 + 1)
            waitset(sets[0])
            compute(sets[0], t2)

            @pl.when(t < NPAIR - 1)
            def _():
                gather(sets[0], t2 + 2)

            waitset(sets[1])
            compute(sets[1], t2 + 1)
            return carry2

        lax.fori_loop(jnp.int32(0), jnp.int32(NPAIR), pair, jnp.int32(0))
        base = r0 * K2
        pltpu.sync_copy(pob, ps.at[pl.ds(base, NBLK * K2)])
        pltpu.sync_copy(nob, ns.at[pl.ds(base, NBLK * K2)])
        return carry

    lax.fori_loop(jnp.int32(0), jnp.int32(OUTER_S), outer, jnp.int32(0))


@functools.cache
def _sim_call():
    mesh = plsc.VectorSubcoreMesh(core_axis_name="c", subcore_axis_name="s")
    return pl.kernel(
        _sim_sc_body,
        out_type=(jax.ShapeDtypeStruct((NEP,), jnp.float32),
                  jax.ShapeDtypeStruct((NEP,), jnp.float32)),
        mesh=mesh,
        compiler_params=pltpu.CompilerParams(use_tc_tiling_on_sc=False, needs_layout_passes=False),
        scratch_types=[
            pltpu.VMEM((NBLK, K2), jnp.int32),
            pltpu.VMEM((NBLK, K2), jnp.int32),
            pltpu.VMEM((NBLK, K2), jnp.int32),
            pltpu.VMEM((K2, DFP), jnp.int32),
            pltpu.VMEM((K2, DFP), jnp.int32),
            pltpu.VMEM((K2, DFP), jnp.int32),
            pltpu.VMEM((K2, DFP), jnp.int32),
            pltpu.VMEM((K2, DFP), jnp.int32),
            pltpu.VMEM((K2, DFP), jnp.int32),
            pltpu.VMEM((NBLK * K2,), jnp.float32),
            pltpu.VMEM((NBLK * K2,), jnp.float32),
            pltpu.SemaphoreType.DMA,
            pltpu.SemaphoreType.DMA,
            pltpu.SemaphoreType.DMA,
            pltpu.SemaphoreType.DMA,
            pltpu.SemaphoreType.DMA,
            pltpu.SemaphoreType.DMA,
        ],
    )


def _t0_body(x_ref, w_ref, dinv_ref, y2_ref):
    xw = jnp.dot(x_ref[...], w_ref[...], preferred_element_type=jnp.float32)
    y = xw * dinv_ref[...]
    y2_ref[0] = y[:, :H]
    y2_ref[1] = y[:, H:]


_t0 = pl.pallas_call(
    _t0_body,
    grid=(GRID_TC,),
    in_specs=[
        pl.BlockSpec((ROWBLK, D), lambda i: (i, _I0)),
        pl.BlockSpec((D, D), lambda i: (_I0, _I0)),
        pl.BlockSpec((ROWBLK, 1), lambda i: (i, _I0)),
    ],
    out_specs=pl.BlockSpec((2, ROWBLK, H), lambda i: (_I0, i, _I0)),
    out_shape=jax.ShapeDtypeStruct((2, N, H), jnp.float32),
)


def _post_layer(acc_ref, y_ref, dinv_ref, b_ref):
    t0 = dinv_ref[...] * (acc_ref[0] + y_ref[0]) + b_ref[0, :H]
    t1 = dinv_ref[...] * (acc_ref[1] + y_ref[1]) + b_ref[0, H:]
    u0 = jnp.where(t0 > 0, t0, 0.01 * t0)
    u1 = jnp.where(t1 > 0, t1, 0.01 * t1)
    nsq = (jnp.sum(u0 * u0, axis=1, keepdims=True)
           + jnp.sum(u1 * u1, axis=1, keepdims=True))
    scale = 1.0 / jnp.maximum(jnp.sqrt(nsq), 1e-12)
    return u0 * scale, u1 * scale


def _fmid_body(acc_ref, y_ref, dinv_ref, b_ref, w_ref, e_ref, yn_ref):
    e0, e1 = _post_layer(acc_ref, y_ref, dinv_ref, b_ref)
    e_ref[:, :H] = e0
    e_ref[:, H:] = e1
    e = jnp.concatenate([e0, e1], axis=1)
    yn = jnp.dot(e, w_ref[...], preferred_element_type=jnp.float32) * dinv_ref[...]
    yn_ref[0] = yn[:, :H]
    yn_ref[1] = yn[:, H:]


_fmid = pl.pallas_call(
    _fmid_body,
    grid=(GRID_TC,),
    in_specs=[
        pl.BlockSpec((2, ROWBLK, H), lambda i: (_I0, i, _I0)),
        pl.BlockSpec((2, ROWBLK, H), lambda i: (_I0, i, _I0)),
        pl.BlockSpec((ROWBLK, 1), lambda i: (i, _I0)),
        pl.BlockSpec((1, D), lambda i: (_I0, _I0)),
        pl.BlockSpec((D, D), lambda i: (_I0, _I0)),
    ],
    out_specs=[
        pl.BlockSpec((ROWBLK, D), lambda i: (i, _I0)),
        pl.BlockSpec((2, ROWBLK, H), lambda i: (_I0, i, _I0)),
    ],
    out_shape=[
        jax.ShapeDtypeStruct((N, D), jnp.float32),
        jax.ShapeDtypeStruct((2, N, H), jnp.float32),
    ],
)


def _flast_body(acc_ref, y_ref, dinv_ref, b_ref, e_ref):
    e0, e1 = _post_layer(acc_ref, y_ref, dinv_ref, b_ref)
    e_ref[:, :H] = e0
    e_ref[:, H:] = e1


_flast = pl.pallas_call(
    _flast_body,
    grid=(GRID_TC,),
    in_specs=[
        pl.BlockSpec((2, ROWBLK, H), lambda i: (_I0, i, _I0)),
        pl.BlockSpec((2, ROWBLK, H), lambda i: (_I0, i, _I0)),
        pl.BlockSpec((ROWBLK, 1), lambda i: (i, _I0)),
        pl.BlockSpec((1, D), lambda i: (_I0, _I0)),
    ],
    out_specs=pl.BlockSpec((ROWBLK, D), lambda i: (i, _I0)),
    out_shape=jax.ShapeDtypeStruct((N, D), jnp.float32),
)


def kernel(users, pos, neg, E, W1, b1, W2, b2, W3, b3):
    E = E.astype(jnp.float32)
    W1, b1, W2, b2, W3, b3 = (t.astype(jnp.float32)
                              for t in (W1, b1, W2, b2, W3, b3))
    u32 = users.astype(jnp.uint32)
    p32 = pos.astype(jnp.uint32)
    nn = jnp.uint32(N)
    eid = jnp.sort(jnp.concatenate([u32 * nn + p32, p32 * nn + u32]))
    keep = jnp.concatenate([jnp.ones((1,), bool), eid[1:] != eid[:-1]])
    src = (eid // nn).astype(jnp.int32)
    dst = (eid % nn).astype(jnp.int32)
    srcS = jnp.where(keep, src, N).astype(jnp.int32)
    dstG = jnp.where(keep, dst, 0).astype(jnp.int32)
    deg = jax.ops.segment_sum(keep.astype(jnp.float32), src, num_segments=N) + 1.0
    dinv = lax.rsqrt(deg)
    dinv2 = dinv[:, None]

    dstG2 = jnp.concatenate([dstG, dstG + N]).reshape(2 * E2K, K)
    srcS2 = srcS.reshape(E2K, K)
    zz = jnp.zeros((NPAD, H), jnp.float32)
    b1r, b2r, b3r = (b.reshape(1, D) for b in (b1, b2, b3))

    def gcn(y2):
        acc = _gcn_call()(y2.reshape(2 * N, H), dstG2, srcS2, zz)
        return acc.reshape(2, NPAD, H)[:, :N, :]

    y2a = _t0(E, W1, dinv2)
    e1, y2b = _fmid(gcn(y2a), y2a, dinv2, b1r, W2)
    e2, y2c = _fmid(gcn(y2b), y2b, dinv2, b2r, W3)
    e3 = _flast(gcn(y2c), y2c, dinv2, b3r)

    ef32 = jnp.concatenate([E, e1, e2, e3], axis=1)
    ef = lax.bitcast_convert_type(
        ef32.astype(jnp.bfloat16).reshape(N, DFP, 2), jnp.int32)
    padw = NEP - NE
    ui = jnp.pad(users.astype(jnp.int32), (0, padw)).reshape(NEK, K2)
    pi = jnp.pad(pos.astype(jnp.int32), (0, padw)).reshape(NEK, K2)
    ni = jnp.pad(neg.astype(jnp.int32), (0, padw)).reshape(NEK, K2)
    ps, ns = _sim_call()(ef, ui, pi, ni)
    return ps[:NE].astype(jnp.float64), ns[:NE].astype(jnp.float64)
